# baseline (speedup 1.0000x reference)
import functools

import jax
import jax.numpy as jnp
from jax import lax
from jax.experimental import pallas as pl
from jax.experimental.pallas import tpu as pltpu
from jax.experimental.pallas import tpu_sc as plsc

NC = 2
NS = 16
NW = NC * NS
LANES = 16
CHUNK_ROWS = 32


def kernel(x, pos_embedding):
    batch, seq_len, emb = x.shape

    seq_per_w = seq_len // NW
    n_chunks = seq_per_w // CHUNK_ROWS
    groups_per_row = emb // (LANES * 8)

    mesh = plsc.VectorSubcoreMesh(core_axis_name="c", subcore_axis_name="s")

    @functools.partial(
        pl.kernel,
        mesh=mesh,
        out_type=jax.ShapeDtypeStruct((batch, seq_len, emb), jnp.float32),
        scratch_types=[
            pltpu.VMEM((CHUNK_ROWS, emb), jnp.float32),
            pltpu.VMEM((CHUNK_ROWS, emb), jnp.float32),
            pltpu.VMEM((CHUNK_ROWS, emb), jnp.float32),
            pltpu.VMEM((CHUNK_ROWS, emb), jnp.float32),
            pltpu.SemaphoreType.DMA,
            pltpu.SemaphoreType.DMA,
            pltpu.SemaphoreType.DMA,
            pltpu.SemaphoreType.DMA,
            pltpu.SemaphoreType.DMA,
            pltpu.SemaphoreType.DMA,
        ],
    )
    def run(x_hbm, pos_hbm, out_hbm, xv0, xv1, pv0, pv1,
            sin0, sin1, sout0, sout1, spos0, spos1):
        wid = lax.axis_index("s") * NC + lax.axis_index("c")
        seq_base = wid * seq_per_w

        xv = [xv0, xv1]
        sin = [sin0, sin1]
        sout = [sout0, sout1]
        pv = [pv0, pv1]
        spos = [spos0, spos1]

        steps = [(g, b) for g in range(n_chunks) for b in range(batch)]

        def row0(g):
            return seq_base + g * CHUNK_ROWS

        def start_pos(g):
            return pltpu.async_copy(
                pos_hbm.at[pl.ds(row0(g), CHUNK_ROWS), :], pv[g % 2], spos[g % 2])

        def start_in(i):
            g, b = steps[i]
            return pltpu.async_copy(
                x_hbm.at[b, pl.ds(row0(g), CHUNK_ROWS), :], xv[i % 2], sin[i % 2])

        def start_out(i):
            g, b = steps[i]
            return pltpu.async_copy(
                xv[i % 2], out_hbm.at[b, pl.ds(row0(g), CHUNK_ROWS), :], sout[i % 2])

        pos_h = {0: start_pos(0)}
        in_h = {0: start_in(0)}
        out_h = {}

        for i, (g, b) in enumerate(steps):
            cur = i % 2
            if b == 0 and g + 1 < n_chunks:
                pos_h[g + 1] = start_pos(g + 1)
            if i + 1 < len(steps):
                if i >= 1:
                    out_h[i - 1].wait()
                in_h[i + 1] = start_in(i + 1)
            in_h[i].wait()
            if b == 0:
                pos_h[g].wait()

            buf = xv[cur]
            pbuf = pv[g % 2]

            def row_body(r, _, buf=buf, pbuf=pbuf):
                def grp_body(c, _, r=r, buf=buf, pbuf=pbuf):
                    base = c * (LANES * 8)
                    for k in range(8):
                        o = base + k * LANES
                        buf[r, pl.ds(o, LANES)] = (
                            buf[r, pl.ds(o, LANES)] + pbuf[r, pl.ds(o, LANES)]
                        )
                    return 0

                lax.fori_loop(0, groups_per_row, grp_body, 0)
                return 0

            lax.fori_loop(0, CHUNK_ROWS, row_body, 0)
            out_h[i] = start_out(i)

        out_h[len(steps) - 2].wait()
        out_h[len(steps) - 1].wait()

    return run(x, pos_embedding)


# speedup vs baseline: 2.7846x; 2.7846x over previous
import functools

import jax
import jax.numpy as jnp
from jax import lax
from jax.experimental import pallas as pl
from jax.experimental.pallas import tpu as pltpu
from jax.experimental.pallas import tpu_sc as plsc

NC = 2
NS = 16
NW = NC * NS
LANES = 16
CHUNK_ROWS = 32


def kernel(x, pos_embedding):
    batch, seq_len, emb = x.shape

    seq_per_w = seq_len // NW
    n_chunks = seq_per_w // CHUNK_ROWS
    groups_per_row = emb // (LANES * 8)

    mesh = plsc.VectorSubcoreMesh(core_axis_name="c", subcore_axis_name="s")

    @functools.partial(
        pl.kernel,
        mesh=mesh,
        out_type=jax.ShapeDtypeStruct((batch, seq_len, emb), jnp.float32),
        scratch_types=[
            pltpu.VMEM((CHUNK_ROWS, emb), jnp.float32),
            pltpu.VMEM((CHUNK_ROWS, emb), jnp.float32),
            pltpu.VMEM((CHUNK_ROWS, emb), jnp.float32),
            pltpu.VMEM((CHUNK_ROWS, emb), jnp.float32),
            pltpu.SemaphoreType.DMA,
            pltpu.SemaphoreType.DMA,
            pltpu.SemaphoreType.DMA,
            pltpu.SemaphoreType.DMA,
            pltpu.SemaphoreType.DMA,
            pltpu.SemaphoreType.DMA,
        ],
    )
    def run(x_hbm, pos_hbm, out_hbm, xv0, xv1, pv0, pv1,
            sin0, sin1, sout0, sout1, spos0, spos1):
        wid = lax.axis_index("s") * NC + lax.axis_index("c")
        seq_base = wid * seq_per_w

        xv = [xv0, xv1]
        sin = [sin0, sin1]
        sout = [sout0, sout1]
        pv = [pv0, pv1]
        spos = [spos0, spos1]

        steps = [(g, b) for g in range(n_chunks) for b in range(batch)]

        def row0(g):
            return seq_base + g * CHUNK_ROWS

        def start_pos(g):
            return pltpu.async_copy(
                pos_hbm.at[pl.ds(row0(g), CHUNK_ROWS), :], pv[g % 2], spos[g % 2])

        def start_in(i):
            g, b = steps[i]
            return pltpu.async_copy(
                x_hbm.at[b, pl.ds(row0(g), CHUNK_ROWS), :], xv[i % 2], sin[i % 2])

        def start_out(i):
            g, b = steps[i]
            return pltpu.async_copy(
                xv[i % 2], out_hbm.at[b, pl.ds(row0(g), CHUNK_ROWS), :], sout[i % 2])

        pos_h = {0: start_pos(0)}
        in_h = {0: start_in(0)}
        out_h = {}

        for i, (g, b) in enumerate(steps):
            cur = i % 2
            if b == 0 and g + 1 < n_chunks:
                pos_h[g + 1] = start_pos(g + 1)
            if i + 1 < len(steps):
                if i >= 1:
                    out_h[i - 1].wait()
                in_h[i + 1] = start_in(i + 1)
            in_h[i].wait()
            if b == 0:
                pos_h[g].wait()

            buf = xv[cur]
            pbuf = pv[g % 2]

            def row_body(r, _, buf=buf, pbuf=pbuf):
                row_x = buf.at[r]
                row_p = pbuf.at[r]

                def grp_body(c, _, row_x=row_x, row_p=row_p):
                    base = c * (LANES * 8)
                    for k in range(8):
                        o = base + k * LANES
                        row_x[pl.ds(o, LANES)] = (
                            row_x[pl.ds(o, LANES)] + row_p[pl.ds(o, LANES)]
                        )
                    return 0

                lax.fori_loop(0, groups_per_row, grp_body, 0)
                return 0

            lax.fori_loop(0, CHUNK_ROWS, row_body, 0)
            out_h[i] = start_out(i)

        out_h[len(steps) - 2].wait()
        out_h[len(steps) - 1].wait()

    return run(x, pos_embedding)


# speedup vs baseline: 3.0262x; 1.0868x over previous
import functools

import jax
import jax.numpy as jnp
from jax import lax
from jax.experimental import pallas as pl
from jax.experimental.pallas import tpu as pltpu
from jax.experimental.pallas import tpu_sc as plsc

NC = 2
NS = 16
NW = NC * NS
LANES = 16
CHUNK_ROWS = 32
XBUFS = 3


def kernel(x, pos_embedding):
    batch, seq_len, emb = x.shape

    seq_per_w = seq_len // NW
    n_chunks = seq_per_w // CHUNK_ROWS
    groups_per_row = emb // (LANES * 8)
    half = CHUNK_ROWS // 2

    mesh = plsc.VectorSubcoreMesh(core_axis_name="c", subcore_axis_name="s")

    @functools.partial(
        pl.kernel,
        mesh=mesh,
        out_type=jax.ShapeDtypeStruct((batch, seq_len, emb), jnp.float32),
        scratch_types=[
            pltpu.VMEM((CHUNK_ROWS, emb), jnp.float32),
            pltpu.VMEM((CHUNK_ROWS, emb), jnp.float32),
            pltpu.VMEM((CHUNK_ROWS, emb), jnp.float32),
            pltpu.VMEM((CHUNK_ROWS, emb), jnp.float32),
            pltpu.VMEM((CHUNK_ROWS, emb), jnp.float32),
            pltpu.SemaphoreType.DMA,
            pltpu.SemaphoreType.DMA,
            pltpu.SemaphoreType.DMA,
            pltpu.SemaphoreType.DMA,
            pltpu.SemaphoreType.DMA,
            pltpu.SemaphoreType.DMA,
            pltpu.SemaphoreType.DMA,
            pltpu.SemaphoreType.DMA,
        ],
    )
    def run(x_hbm, pos_hbm, out_hbm, xv0, xv1, xv2, pv0, pv1,
            sin0, sin1, sin2, sout0, sout1, sout2, spos0, spos1):
        wid = lax.axis_index("s") * NC + lax.axis_index("c")
        seq_base = wid * seq_per_w

        xv = [xv0, xv1, xv2]
        sin = [sin0, sin1, sin2]
        sout = [sout0, sout1, sout2]
        pv = [pv0, pv1]
        spos = [spos0, spos1]

        steps = [(g, b) for g in range(n_chunks) for b in range(batch)]
        n_steps = len(steps)

        def row0(g):
            return seq_base + g * CHUNK_ROWS

        def start_pos(g):
            return pltpu.async_copy(
                pos_hbm.at[pl.ds(row0(g), CHUNK_ROWS), :], pv[g % 2], spos[g % 2])

        def start_in(i):
            g, b = steps[i]
            return pltpu.async_copy(
                x_hbm.at[b, pl.ds(row0(g), CHUNK_ROWS), :], xv[i % XBUFS],
                sin[i % XBUFS])

        def start_out(i):
            g, b = steps[i]
            return pltpu.async_copy(
                xv[i % XBUFS], out_hbm.at[b, pl.ds(row0(g), CHUNK_ROWS), :],
                sout[i % XBUFS])

        pos_h = {0: start_pos(0)}
        in_h = {0: start_in(0), 1: start_in(1)}
        out_h = {}

        for i, (g, b) in enumerate(steps):
            cur = i % XBUFS
            if b == 0 and g + 1 < n_chunks:
                pos_h[g + 1] = start_pos(g + 1)
            if i + 2 < n_steps:
                if i >= 1:
                    out_h.pop(i - 1).wait()
                in_h[i + 2] = start_in(i + 2)
            in_h[i].wait()
            if b == 0:
                pos_h[g].wait()

            buf = xv[cur]
            pbuf = pv[g % 2]

            def row_body(r, _, buf=buf, pbuf=pbuf):
                row_x = buf.at[r]
                row_p = pbuf.at[r]

                def grp_body(c, _, row_x=row_x, row_p=row_p):
                    base = c * (LANES * 8)
                    for k in range(8):
                        o = base + k * LANES
                        row_x[pl.ds(o, LANES)] = (
                            row_x[pl.ds(o, LANES)] + row_p[pl.ds(o, LANES)]
                        )
                    return 0

                lax.fori_loop(0, groups_per_row, grp_body, 0)
                return 0

            lax.fori_loop(0, CHUNK_ROWS, row_body, 0)
            out_h[i] = start_out(i)

        for i in (n_steps - 2, n_steps - 1):
            out_h.pop(i).wait()

    return run(x, pos_embedding)


# speedup vs baseline: 3.0325x; 1.0021x over previous
import functools

import jax
import jax.numpy as jnp
from jax import lax
from jax.experimental import pallas as pl
from jax.experimental.pallas import tpu as pltpu
from jax.experimental.pallas import tpu_sc as plsc

NC = 2
NS = 16
NW = NC * NS
LANES = 16
CHUNK_ROWS = 32
XBUFS = 3


def kernel(x, pos_embedding):
    batch, seq_len, emb = x.shape

    seq_per_w = seq_len // NW
    n_chunks = seq_per_w // CHUNK_ROWS
    groups_per_row = emb // (LANES * 8)
    half = CHUNK_ROWS // 2

    mesh = plsc.VectorSubcoreMesh(core_axis_name="c", subcore_axis_name="s")

    @functools.partial(
        pl.kernel,
        mesh=mesh,
        out_type=jax.ShapeDtypeStruct((batch, seq_len, emb), jnp.float32),
        scratch_types=[
            pltpu.VMEM((CHUNK_ROWS, emb), jnp.float32),
            pltpu.VMEM((CHUNK_ROWS, emb), jnp.float32),
            pltpu.VMEM((CHUNK_ROWS, emb), jnp.float32),
            pltpu.VMEM((CHUNK_ROWS, emb), jnp.float32),
            pltpu.VMEM((CHUNK_ROWS, emb), jnp.float32),
            pltpu.SemaphoreType.DMA,
            pltpu.SemaphoreType.DMA,
            pltpu.SemaphoreType.DMA,
            pltpu.SemaphoreType.DMA,
            pltpu.SemaphoreType.DMA,
            pltpu.SemaphoreType.DMA,
            pltpu.SemaphoreType.DMA,
            pltpu.SemaphoreType.DMA,
        ],
    )
    def run(x_hbm, pos_hbm, out_hbm, xv0, xv1, xv2, pv0, pv1,
            sin0, sin1, sin2, sout0, sout1, sout2, spos0, spos1):
        wid = lax.axis_index("s") * NC + lax.axis_index("c")
        seq_base = wid * seq_per_w

        xv = [xv0, xv1, xv2]
        sin = [sin0, sin1, sin2]
        sout = [sout0, sout1, sout2]
        pv = [pv0, pv1]
        spos = [spos0, spos1]

        steps = [(g, b) for g in range(n_chunks) for b in range(batch)]
        n_steps = len(steps)

        def row0(g):
            return seq_base + g * CHUNK_ROWS

        def start_pos(g):
            return pltpu.async_copy(
                pos_hbm.at[pl.ds(row0(g), CHUNK_ROWS), :], pv[g % 2], spos[g % 2])

        def start_in(i):
            g, b = steps[i]
            return pltpu.async_copy(
                x_hbm.at[b, pl.ds(row0(g), CHUNK_ROWS), :], xv[i % XBUFS],
                sin[i % XBUFS])

        def start_out_half(i, h):
            g, b = steps[i]
            return pltpu.async_copy(
                xv[i % XBUFS].at[pl.ds(h * half, half), :],
                out_hbm.at[b, pl.ds(row0(g) + h * half, half), :],
                sout[i % XBUFS])

        def wait_out(i):
            g, b = steps[i]
            for h in range(2):
                pltpu.make_async_copy(
                    xv[i % XBUFS].at[pl.ds(h * half, half), :],
                    out_hbm.at[b, pl.ds(row0(g) + h * half, half), :],
                    sout[i % XBUFS]).wait()

        pos_h = {0: start_pos(0)}
        in_h = {0: start_in(0), 1: start_in(1)}
        out_h = {}

        for i, (g, b) in enumerate(steps):
            cur = i % XBUFS
            if b == 0 and g + 1 < n_chunks:
                pos_h[g + 1] = start_pos(g + 1)
            if i + 2 < n_steps:
                if i >= 1:
                    wait_out(i - 1)
                in_h[i + 2] = start_in(i + 2)
            in_h[i].wait()
            if b == 0:
                pos_h[g].wait()

            buf = xv[cur]
            pbuf = pv[g % 2]

            def row_body(r, _, buf=buf, pbuf=pbuf, i=i):
                row_x = buf.at[r]
                row_p = pbuf.at[r]

                def grp_body(c, _, row_x=row_x, row_p=row_p):
                    base = c * (LANES * 8)
                    for k in range(8):
                        o = base + k * LANES
                        row_x[pl.ds(o, LANES)] = (
                            row_x[pl.ds(o, LANES)] + row_p[pl.ds(o, LANES)]
                        )
                    return 0

                lax.fori_loop(0, groups_per_row, grp_body, 0)

                @pl.when(r == half - 1)
                def _(i=i):
                    start_out_half(i, 0)

                return 0

            lax.fori_loop(0, CHUNK_ROWS, row_body, 0)
            start_out_half(i, 1)

        for i in (n_steps - 3, n_steps - 2, n_steps - 1):
            wait_out(i)

    return run(x, pos_embedding)
